# baseline (speedup 1.0000x reference)
import functools

import jax
import jax.numpy as jnp
from jax import lax
from jax.experimental import pallas as pl
from jax.experimental.pallas import tpu as pltpu
from jax.experimental.pallas import tpu_sc as plsc

N = 10000
XD = 128
HD = 64
ZD = 32
E = 320000

NCORES = 2
NSUB = 16
NTILES = NCORES * NSUB
CHUNK = 128
CPT = 80
EPAD = NTILES * CPT * CHUNK
NP = 10240
RPT = NP // NSUB
NBUF = 8

TB = 2000
GRID = N // TB


def _sc_scatter_body(W, x_hbm, s_hbm, d_hbm, out_hbm,
                     s_v, d_v, *rest):
    c = lax.axis_index("c")
    t = lax.axis_index("s")
    wid = t * NCORES + c
    bufs = rest[:NBUF]
    acc = rest[NBUF]
    gsems = rest[NBUF + 1:2 * NBUF + 1]
    ssems = rest[2 * NBUF + 1:3 * NBUF + 1]
    semz = rest[3 * NBUF + 1]
    b0 = bufs[0]

    z16 = jnp.zeros((16,), jnp.float32)

    def _zb(i, carry):
        for j in range(W // 16):
            b0[i, pl.ds(j * 16, 16)] = z16
        return carry

    lax.fori_loop(0, CHUNK, _zb, 0)
    zh = []
    for kk in range(RPT // CHUNK):
        zh.append(pltpu.async_copy(
            b0, acc.at[pl.ds(t * RPT + kk * CHUNK, CHUNK)], semz))
    for h in zh:
        h.wait()
    plsc.subcore_barrier()

    pltpu.sync_copy(s_hbm.at[wid], s_v)
    pltpu.sync_copy(d_hbm.at[wid], d_v)

    gh0 = []
    for b in range(NBUF):
        gh0.append(pltpu.async_copy(x_hbm.at[s_v.at[b]], bufs[b], gsems[b]))
    for b in range(NBUF):
        gh0[b].wait()
        pltpu.async_copy(bufs[b], acc.at[d_v.at[b]], ssems[b], add=True)

    def _grp(g, carry):
        base = g * NBUF
        ghs = []
        for b in range(NBUF):
            pltpu.make_async_copy(
                bufs[b], acc.at[d_v.at[base + b]], ssems[b]).wait()
            ghs.append(pltpu.async_copy(
                x_hbm.at[s_v.at[base + b]], bufs[b], gsems[b]))
        for b in range(NBUF):
            ghs[b].wait()
            pltpu.async_copy(
                bufs[b], acc.at[d_v.at[base + b]], ssems[b], add=True)
        return carry

    lax.fori_loop(1, CPT // NBUF, _grp, 0)
    for b in range(NBUF):
        pltpu.make_async_copy(bufs[b], acc.at[d_v.at[b]], ssems[b]).wait()
    plsc.subcore_barrier()

    pltpu.sync_copy(acc.at[pl.ds(t * RPT, RPT)],
                    out_hbm.at[c, pl.ds(t * RPT, RPT)])


@functools.lru_cache(maxsize=None)
def _make_sc_scatter(W):
    body = functools.partial(_sc_scatter_body, W)
    mesh = plsc.VectorSubcoreMesh(core_axis_name="c", subcore_axis_name="s")
    return pl.kernel(
        body,
        out_type=jax.ShapeDtypeStruct((NCORES, NP, W), jnp.float32),
        mesh=mesh,
        compiler_params=pltpu.CompilerParams(use_tc_tiling_on_sc=False),
        scratch_types=(
            [pltpu.VMEM((CPT, CHUNK), jnp.int32),
             pltpu.VMEM((CPT, CHUNK), jnp.int32)]
            + [pltpu.VMEM((CHUNK, W), jnp.float32)] * NBUF
            + [pltpu.VMEM_SHARED((NP, W), jnp.float32)]
            + [pltpu.SemaphoreType.DMA] * (2 * NBUF + 1)
        ),
    )


def _sc_deg_body(d_hbm, out_hbm, d_v, ones_v, zb, acc, semz):
    c = lax.axis_index("c")
    t = lax.axis_index("s")
    wid = t * NCORES + c

    one16 = jnp.ones((16,), jnp.float32)
    z16 = jnp.zeros((16,), jnp.float32)

    def _fill(i, carry):
        ones_v[pl.ds(i * 16, 16)] = one16
        return carry

    lax.fori_loop(0, CHUNK // 16, _fill, 0)

    def _zb(i, carry):
        zb[pl.ds(i * 16, 16)] = z16
        return carry

    lax.fori_loop(0, RPT // 16, _zb, 0)
    pltpu.async_copy(zb, acc.at[pl.ds(t * RPT, RPT)], semz).wait()
    plsc.subcore_barrier()

    pltpu.sync_copy(d_hbm.at[wid], d_v)

    def _step(j, carry):
        pltpu.async_copy(ones_v, acc.at[d_v.at[j]], semz, add=True)
        return carry

    lax.fori_loop(0, CPT, _step, 0)

    def _drain(j, carry):
        pltpu.make_async_copy(ones_v, acc.at[d_v.at[j]], semz).wait()
        return carry

    lax.fori_loop(0, CPT, _drain, 0)
    plsc.subcore_barrier()

    pltpu.sync_copy(acc.at[pl.ds(t * RPT, RPT)],
                    out_hbm.at[c, pl.ds(t * RPT, RPT)])


@functools.lru_cache(maxsize=None)
def _make_sc_deg():
    mesh = plsc.VectorSubcoreMesh(core_axis_name="c", subcore_axis_name="s")
    return pl.kernel(
        _sc_deg_body,
        out_type=jax.ShapeDtypeStruct((NCORES, NP), jnp.float32),
        mesh=mesh,
        compiler_params=pltpu.CompilerParams(use_tc_tiling_on_sc=False),
        scratch_types=[
            pltpu.VMEM((CPT, CHUNK), jnp.int32),
            pltpu.VMEM((CHUNK,), jnp.float32),
            pltpu.VMEM((RPT,), jnp.float32),
            pltpu.VMEM_SHARED((NP,), jnp.float32),
            pltpu.SemaphoreType.DMA,
        ],
    )


def _sc_deg(d3):
    return _make_sc_deg()(d3)


def _sc_scatter64(x, s3, d3):
    return _make_sc_scatter(HD)(x, s3, d3)


def _sc_scatter32(x, s3, d3):
    return _make_sc_scatter(ZD)(x, s3, d3)


def _mm(a, w):
    return jnp.dot(a, w, preferred_element_type=jnp.float32)


def _row(w):
    return pl.BlockSpec((TB, w), lambda i: (i, 0))


def _Ssp(w):
    return pl.BlockSpec((NCORES, TB, w), lambda i: (0, i, 0))


def _whole(shape):
    return pl.BlockSpec(shape, lambda i, _n=len(shape): (0,) * _n)


def _row0(w):
    return pl.BlockSpec((8, w), lambda i: (0, 0))


def _tc(body, out_shapes, in_specs, out_specs, *args):
    return pl.pallas_call(
        body, out_shape=out_shapes, grid=(GRID,),
        in_specs=in_specs, out_specs=out_specs,
        compiler_params=pltpu.CompilerParams(
            dimension_semantics=("arbitrary",)),
    )(*args)


def _prep_body(degp_ref, xs_ref, w_ref, b_ref, dinv_ref, xp_ref, xps_ref):
    deg = degp_ref[0] + degp_ref[1] + 1.0
    dinv = lax.rsqrt(deg)
    dinv_ref[...] = dinv
    xp = jnp.maximum(_mm(xs_ref[...], w_ref[...]) + b_ref[...][None, :], 0.0)
    xp_ref[...] = xp
    xps_ref[...] = xp * dinv


def _axp_body(S_ref, xp_ref, dinv_ref, wc_ref, bc_ref, wz_ref, bz_ref,
              wr_ref, br_ref, wh_ref, bh_ref, bp_ref, wpm_ref, bpm_ref,
              c1a_ref, xza_ref, xra_ref, xha_ref, z0_ref):
    dinv = dinv_ref[...]
    axp = dinv * (S_ref[0] + S_ref[1]) + (dinv * dinv) * xp_ref[...]
    c1a_ref[...] = _mm(axp, wc_ref[...]) + bc_ref[...][None, :]
    xza_ref[...] = _mm(axp, wz_ref[...]) + bz_ref[...][None, :]
    xra_ref[...] = _mm(axp, wr_ref[...]) + br_ref[...][None, :]
    xha_ref[...] = _mm(axp, wh_ref[...]) + bh_ref[...][None, :]
    pr0 = jnp.maximum(bp_ref[...], 0.0)[None, :]
    z0row = _mm(pr0, wpm_ref[...]) + bpm_ref[...][None, :]
    z0_ref[...] = jnp.broadcast_to(z0row, (TB, ZD))


def _b0_body(c1a_ref, dinv_ref, wm_ref, y2_ref, y2s_ref):
    hc = jnp.maximum(c1a_ref[...], 0.0)
    y2 = _mm(hc, wm_ref[...])
    y2_ref[...] = y2
    y2s_ref[...] = y2 * dinv_ref[...]


def _b_body(S_ref, h_ref, dinv_ref, c1a_ref, wc_ref, wm_ref,
            whz_ref, bhz_ref, whr_ref, bhr_ref,
            y2_ref, y2s_ref, hz_ref, hr_ref):
    dinv = dinv_ref[...]
    h = h_ref[...]
    ah = dinv * (S_ref[0] + S_ref[1]) + (dinv * dinv) * h
    hc = jnp.maximum(c1a_ref[...] + _mm(ah, wc_ref[...]), 0.0)
    y2 = _mm(hc, wm_ref[...])
    y2_ref[...] = y2
    y2s_ref[...] = y2 * dinv
    hz_ref[...] = _mm(ah, whz_ref[...]) + bhz_ref[...][None, :]
    hr_ref[...] = _mm(ah, whr_ref[...]) + bhr_ref[...][None, :]


def _c_body(S_ref, y2_ref, dinv_ref, bm_ref, wpz_ref, bpz_ref,
            phi_ref, phis_ref):
    dinv = dinv_ref[...]
    z_enc = dinv * (S_ref[0] + S_ref[1]) \
        + (dinv * dinv) * y2_ref[...] + bm_ref[...][None, :]
    phi = jnp.maximum(_mm(z_enc, wpz_ref[...]) + bpz_ref[...][None, :], 0.0)
    phi_ref[...] = phi
    phis_ref[...] = phi * dinv


def _d0a_body(S_ref, phi_ref, dinv_ref, xza_ref, xha_ref,
              wzb_ref, whb_ref, bhz_ref, bhh_ref,
              wpr_ref, bpr_ref, wpm_ref, bpm_ref,
              h_ref, z_ref, hs_ref):
    dinv = dinv_ref[...]
    aphi = dinv * (S_ref[0] + S_ref[1]) + (dinv * dinv) * phi_ref[...]
    z_g = jax.nn.sigmoid(xza_ref[...] + _mm(aphi, wzb_ref[...])
                         + bhz_ref[...][None, :])
    xh = xha_ref[...] + _mm(aphi, whb_ref[...])
    h_hat = jnp.tanh(xh + bhh_ref[...][None, :])
    h = (1.0 - z_g) * h_hat
    h_ref[...] = h
    prior = jnp.maximum(_mm(h, wpr_ref[...]) + bpr_ref[...][None, :], 0.0)
    z_ref[...] = _mm(prior, wpm_ref[...]) + bpm_ref[...][None, :]
    hs_ref[...] = h * dinv


def _d_body(S_ref, phi_ref, dinv_ref, xza_ref, xra_ref, xha_ref,
            wzb_ref, wrb_ref, whb_ref, hz_ref, hr_ref, h_ref, whh_ref,
            zg_ref, xh_ref, y4_ref, y4s_ref):
    dinv = dinv_ref[...]
    aphi = dinv * (S_ref[0] + S_ref[1]) + (dinv * dinv) * phi_ref[...]
    z_g = jax.nn.sigmoid(xza_ref[...] + _mm(aphi, wzb_ref[...]) + hz_ref[...])
    r_g = jax.nn.sigmoid(xra_ref[...] + _mm(aphi, wrb_ref[...]) + hr_ref[...])
    zg_ref[...] = z_g
    xh_ref[...] = xha_ref[...] + _mm(aphi, whb_ref[...])
    y4 = _mm(r_g * h_ref[...], whh_ref[...])
    y4_ref[...] = y4
    y4s_ref[...] = y4 * dinv


def _ea_body(S_ref, y4_ref, dinv_ref, bhh_ref, xh_ref, zg_ref, hold_ref,
             wpr_ref, bpr_ref, wpm_ref, bpm_ref,
             h_ref, z_ref, hs_ref):
    dinv = dinv_ref[...]
    hh = dinv * (S_ref[0] + S_ref[1]) \
        + (dinv * dinv) * y4_ref[...] + bhh_ref[...][None, :]
    h_hat = jnp.tanh(xh_ref[...] + hh)
    z_g = zg_ref[...]
    h = z_g * hold_ref[0:1, :] + (1.0 - z_g) * h_hat
    h_ref[...] = h
    prior = jnp.maximum(_mm(h, wpr_ref[...]) + bpr_ref[...][None, :], 0.0)
    z_ref[...] = _mm(prior, wpm_ref[...]) + bpm_ref[...][None, :]
    hs_ref[...] = h * dinv


def _e_body(S_ref, y4_ref, dinv_ref, bhh_ref, xh_ref, zg_ref, hold_ref,
            h_ref):
    dinv = dinv_ref[...]
    hh = dinv * (S_ref[0] + S_ref[1]) \
        + (dinv * dinv) * y4_ref[...] + bhh_ref[...][None, :]
    h_hat = jnp.tanh(xh_ref[...] + hh)
    z_g = zg_ref[...]
    h_ref[...] = z_g * hold_ref[0:1, :] + (1.0 - z_g) * h_hat


_f32 = jnp.float32


def _sds(*shape):
    return jax.ShapeDtypeStruct(shape, _f32)


def kernel(xs, edge_index, W_phi_x, b_phi_x, W_prior, b_prior, W_pmean,
           b_pmean, W_pstd, b_pstd, W_c1, b_c1, W_mean, b_mean, W_phi_z,
           b_phi_z, W_xz, b_xz, W_hz, b_hz, W_xr, b_xr, W_hr, b_hr, W_xh,
           b_xh, W_hh, b_hh):
    s = edge_index[0]
    d = edge_index[1]
    pad = EPAD - E
    s3 = jnp.concatenate([s, jnp.zeros((pad,), jnp.int32)]) \
        .reshape(NTILES, CPT, CHUNK)
    d3 = jnp.concatenate([d, jnp.full((pad,), N, jnp.int32)]) \
        .reshape(NTILES, CPT, CHUNK)

    W_c1a, W_c1b = W_c1[:HD], W_c1[HD:]
    W_xza, W_xzb = W_xz[:HD], W_xz[HD:]
    W_xra, W_xrb = W_xr[:HD], W_xr[HD:]
    W_xha, W_xhb = W_xh[:HD], W_xh[HD:]

    w64 = _whole((HD, HD))
    w6432 = _whole((HD, ZD))
    w3264 = _whole((ZD, HD))
    b64 = _whole((HD,))
    b32 = _whole((ZD,))

    degp = _sc_deg(d3).reshape(NCORES, NP, 1)
    dinv, xp, xps = _tc(
        _prep_body, [_sds(N, 1), _sds(N, HD), _sds(N, HD)],
        [_Ssp(1), _row(XD), _whole((XD, HD)), b64],
        [_row(1), _row(HD), _row(HD)],
        degp, xs, W_phi_x, b_phi_x)

    Sxp = _sc_scatter64(xps, s3, d3)
    C1a, XZa, XRa, XHa, z0 = _tc(
        _axp_body,
        [_sds(N, HD), _sds(N, HD), _sds(N, HD), _sds(N, HD), _sds(N, ZD)],
        [_Ssp(HD), _row(HD), _row(1), w64, b64, w64, b64, w64, b64,
         w64, b64, b64, w6432, b32],
        [_row(HD), _row(HD), _row(HD), _row(HD), _row(ZD)],
        Sxp, xp, dinv, W_c1a, b_c1, W_xza, b_xz, W_xra, b_xr, W_xha, b_xh,
        b_prior, W_pmean, b_pmean)

    y2, y2s = _tc(
        _b0_body, [_sds(N, ZD), _sds(N, ZD)],
        [_row(HD), _row(1), w6432],
        [_row(ZD), _row(ZD)],
        C1a, dinv, W_mean)
    Sz = _sc_scatter32(y2s, s3, d3)
    phi, phis = _tc(
        _c_body, [_sds(N, HD), _sds(N, HD)],
        [_Ssp(ZD), _row(ZD), _row(1), b32, w3264, b64],
        [_row(HD), _row(HD)],
        Sz, y2, dinv, b_mean, W_phi_z, b_phi_z)
    Sphi = _sc_scatter64(phis, s3, d3)
    h, z1, hs = _tc(
        _d0a_body, [_sds(N, HD), _sds(N, ZD), _sds(N, HD)],
        [_Ssp(HD), _row(HD), _row(1), _row(HD), _row(HD),
         w64, w64, b64, b64, w64, b64, w6432, b32],
        [_row(HD), _row(ZD), _row(HD)],
        Sphi, phi, dinv, XZa, XHa, W_xzb, W_xhb, b_hz, b_hh,
        W_prior, b_prior, W_pmean, b_pmean)

    zs = [z0, z1]
    for t in (1, 2):
        Sh = _sc_scatter64(hs, s3, d3)
        y2, y2s, hz, hr = _tc(
            _b_body, [_sds(N, ZD), _sds(N, ZD), _sds(N, HD), _sds(N, HD)],
            [_Ssp(HD), _row(HD), _row(1), _row(HD), w64, w6432,
             w64, b64, w64, b64],
            [_row(ZD), _row(ZD), _row(HD), _row(HD)],
            Sh, h, dinv, C1a, W_c1b, W_mean, W_hz, b_hz, W_hr, b_hr)
        Sz = _sc_scatter32(y2s, s3, d3)
        phi, phis = _tc(
            _c_body, [_sds(N, HD), _sds(N, HD)],
            [_Ssp(ZD), _row(ZD), _row(1), b32, w3264, b64],
            [_row(HD), _row(HD)],
            Sz, y2, dinv, b_mean, W_phi_z, b_phi_z)
        Sphi = _sc_scatter64(phis, s3, d3)
        zg, xh, y4, y4s = _tc(
            _d_body, [_sds(N, HD), _sds(N, HD), _sds(N, HD), _sds(N, HD)],
            [_Ssp(HD), _row(HD), _row(1), _row(HD), _row(HD), _row(HD),
             w64, w64, w64, _row(HD), _row(HD), _row(HD), w64],
            [_row(HD), _row(HD), _row(HD), _row(HD)],
            Sphi, phi, dinv, XZa, XRa, XHa, W_xzb, W_xrb, W_xhb,
            hz, hr, h, W_hh)
        Su = _sc_scatter64(y4s, s3, d3)
        if t < 2:
            h, z_t, hs = _tc(
                _ea_body, [_sds(N, HD), _sds(N, ZD), _sds(N, HD)],
                [_Ssp(HD), _row(HD), _row(1), b64, _row(HD), _row(HD),
                 _row0(HD), w64, b64, w6432, b32],
                [_row(HD), _row(ZD), _row(HD)],
                Su, y4, dinv, b_hh, xh, zg, h,
                W_prior, b_prior, W_pmean, b_pmean)
            zs.append(z_t)
        else:
            h = _tc(
                _e_body, _sds(N, HD),
                [_Ssp(HD), _row(HD), _row(1), b64, _row(HD), _row(HD),
                 _row0(HD)],
                _row(HD),
                Su, y4, dinv, b_hh, xh, zg, h)

    return jnp.stack(zs), h


# speedup vs baseline: 1.0791x; 1.0791x over previous
import functools

import jax
import jax.numpy as jnp
from jax import lax
from jax.experimental import pallas as pl
from jax.experimental.pallas import tpu as pltpu
from jax.experimental.pallas import tpu_sc as plsc

N = 10000
XD = 128
HD = 64
ZD = 32
E = 320000

NCORES = 2
NSUB = 16
NTILES = NCORES * NSUB
CHUNK = 128
CPT = 80
EPAD = NTILES * CPT * CHUNK
NP = 10240
RPT = NP // NSUB
NBUF = 8
_DO_GATHER = True
_DO_SCATTER = True
XRPT = N // NSUB

TB = 2000
GRID = N // TB


def _sc_scatter_body(W, x_hbm, s_hbm, d_hbm, out_hbm,
                     s_v, d_v, *rest):
    c = lax.axis_index("c")
    t = lax.axis_index("s")
    wid = t * NCORES + c
    bufs = rest[:NBUF]
    acc = rest[NBUF]
    gsems = rest[NBUF + 1:2 * NBUF + 1]
    ssems = rest[2 * NBUF + 1:3 * NBUF + 1]
    semz = rest[3 * NBUF + 1]
    b0 = bufs[0]

    z16 = jnp.zeros((16,), jnp.float32)

    def _zb(i, carry):
        for j in range(W // 16):
            b0[i, pl.ds(j * 16, 16)] = z16
        return carry

    lax.fori_loop(0, CHUNK, _zb, 0)
    zh = []
    for kk in range(RPT // CHUNK):
        zh.append(pltpu.async_copy(
            b0, acc.at[pl.ds(t * RPT + kk * CHUNK, CHUNK)], semz))
    for h in zh:
        h.wait()
    plsc.subcore_barrier()

    pltpu.sync_copy(s_hbm.at[wid], s_v)
    pltpu.sync_copy(d_hbm.at[wid], d_v)

    gh0 = []
    for b in range(NBUF):
        gh0.append(pltpu.async_copy(x_hbm.at[s_v.at[b]], bufs[b], gsems[b]))
    for b in range(NBUF):
        gh0[b].wait()
        if _DO_SCATTER:
            pltpu.async_copy(bufs[b], acc.at[d_v.at[b]], ssems[b], add=True)

    def _grp(g, carry):
        base = g * NBUF
        ghs = []
        for b in range(NBUF):
            if _DO_SCATTER:
                pltpu.make_async_copy(
                    bufs[b], acc.at[d_v.at[base + b]], ssems[b]).wait()
            if _DO_GATHER:
                ghs.append(pltpu.async_copy(
                    x_hbm.at[s_v.at[base + b]], bufs[b], gsems[b]))
        for b in range(NBUF):
            if _DO_GATHER:
                ghs[b].wait()
            if _DO_SCATTER:
                pltpu.async_copy(
                    bufs[b], acc.at[d_v.at[base + b]], ssems[b], add=True)
        return carry

    lax.fori_loop(1, CPT // NBUF, _grp, 0)
    if _DO_SCATTER:
        for b in range(NBUF):
            pltpu.make_async_copy(bufs[b], acc.at[d_v.at[b]], ssems[b]).wait()
    plsc.subcore_barrier()

    pltpu.sync_copy(acc.at[pl.ds(t * RPT, RPT)],
                    out_hbm.at[c, pl.ds(t * RPT, RPT)])


@functools.lru_cache(maxsize=None)
def _make_sc_scatter(W):
    body = functools.partial(_sc_scatter_body, W)
    mesh = plsc.VectorSubcoreMesh(core_axis_name="c", subcore_axis_name="s")
    return pl.kernel(
        body,
        out_type=jax.ShapeDtypeStruct((NCORES, NP, W), jnp.float32),
        mesh=mesh,
        compiler_params=pltpu.CompilerParams(use_tc_tiling_on_sc=False),
        scratch_types=(
            [pltpu.VMEM((CPT, CHUNK), jnp.int32),
             pltpu.VMEM((CPT, CHUNK), jnp.int32)]
            + [pltpu.VMEM((CHUNK, W), jnp.float32)] * NBUF
            + [pltpu.VMEM_SHARED((NP, W), jnp.float32)]
            + [pltpu.SemaphoreType.DMA] * (2 * NBUF + 1)
        ),
    )


def _sc_deg_body(d_hbm, out_hbm, d_v, ones_v, zb, acc, semz):
    c = lax.axis_index("c")
    t = lax.axis_index("s")
    wid = t * NCORES + c

    one16 = jnp.ones((16,), jnp.float32)
    z16 = jnp.zeros((16,), jnp.float32)

    def _fill(i, carry):
        ones_v[pl.ds(i * 16, 16)] = one16
        return carry

    lax.fori_loop(0, CHUNK // 16, _fill, 0)

    def _zb(i, carry):
        zb[pl.ds(i * 16, 16)] = z16
        return carry

    lax.fori_loop(0, RPT // 16, _zb, 0)
    pltpu.async_copy(zb, acc.at[pl.ds(t * RPT, RPT)], semz).wait()
    plsc.subcore_barrier()

    pltpu.sync_copy(d_hbm.at[wid], d_v)

    def _step(j, carry):
        pltpu.async_copy(ones_v, acc.at[d_v.at[j]], semz, add=True)
        return carry

    lax.fori_loop(0, CPT, _step, 0)

    def _drain(j, carry):
        pltpu.make_async_copy(ones_v, acc.at[d_v.at[j]], semz).wait()
        return carry

    lax.fori_loop(0, CPT, _drain, 0)
    plsc.subcore_barrier()

    pltpu.sync_copy(acc.at[pl.ds(t * RPT, RPT)],
                    out_hbm.at[c, pl.ds(t * RPT, RPT)])


@functools.lru_cache(maxsize=None)
def _make_sc_deg():
    mesh = plsc.VectorSubcoreMesh(core_axis_name="c", subcore_axis_name="s")
    return pl.kernel(
        _sc_deg_body,
        out_type=jax.ShapeDtypeStruct((NCORES, NP), jnp.float32),
        mesh=mesh,
        compiler_params=pltpu.CompilerParams(use_tc_tiling_on_sc=False),
        scratch_types=[
            pltpu.VMEM((CPT, CHUNK), jnp.int32),
            pltpu.VMEM((CHUNK,), jnp.float32),
            pltpu.VMEM((RPT,), jnp.float32),
            pltpu.VMEM_SHARED((NP,), jnp.float32),
            pltpu.SemaphoreType.DMA,
        ],
    )


def _sc_deg(d3):
    return _make_sc_deg()(d3)


def _sc_scatter64(x, s3, d3):
    return _make_sc_scatter(HD)(x, s3, d3)


def _sc_scatter32(x, s3, d3):
    return _make_sc_scatter(ZD)(x, s3, d3)


def _mm(a, w):
    return jnp.dot(a, w, preferred_element_type=jnp.float32)


def _row(w):
    return pl.BlockSpec((TB, w), lambda i: (i, 0))


def _Ssp(w):
    return pl.BlockSpec((NCORES, TB, w), lambda i: (0, i, 0))


def _whole(shape):
    return pl.BlockSpec(shape, lambda i, _n=len(shape): (0,) * _n)


def _row0(w):
    return pl.BlockSpec((8, w), lambda i: (0, 0))


def _tc(body, out_shapes, in_specs, out_specs, *args):
    return pl.pallas_call(
        body, out_shape=out_shapes, grid=(GRID,),
        in_specs=in_specs, out_specs=out_specs,
        compiler_params=pltpu.CompilerParams(
            dimension_semantics=("arbitrary",)),
    )(*args)


def _prep_body(degp_ref, xs_ref, w_ref, b_ref, dinv_ref, xp_ref, xps_ref):
    deg = degp_ref[0] + degp_ref[1] + 1.0
    dinv = lax.rsqrt(deg)
    dinv_ref[...] = dinv
    xp = jnp.maximum(_mm(xs_ref[...], w_ref[...]) + b_ref[...][None, :], 0.0)
    xp_ref[...] = xp
    xps_ref[...] = xp * dinv


def _axp_body(S_ref, xp_ref, dinv_ref, wc_ref, bc_ref, wz_ref, bz_ref,
              wr_ref, br_ref, wh_ref, bh_ref, bp_ref, wpm_ref, bpm_ref,
              c1a_ref, xza_ref, xra_ref, xha_ref, z0_ref):
    dinv = dinv_ref[...]
    axp = dinv * (S_ref[0] + S_ref[1]) + (dinv * dinv) * xp_ref[...]
    c1a_ref[...] = _mm(axp, wc_ref[...]) + bc_ref[...][None, :]
    xza_ref[...] = _mm(axp, wz_ref[...]) + bz_ref[...][None, :]
    xra_ref[...] = _mm(axp, wr_ref[...]) + br_ref[...][None, :]
    xha_ref[...] = _mm(axp, wh_ref[...]) + bh_ref[...][None, :]
    pr0 = jnp.maximum(bp_ref[...], 0.0)[None, :]
    z0row = _mm(pr0, wpm_ref[...]) + bpm_ref[...][None, :]
    z0_ref[...] = jnp.broadcast_to(z0row, (TB, ZD))


def _b0_body(c1a_ref, dinv_ref, wm_ref, y2_ref, y2s_ref):
    hc = jnp.maximum(c1a_ref[...], 0.0)
    y2 = _mm(hc, wm_ref[...])
    y2_ref[...] = y2
    y2s_ref[...] = y2 * dinv_ref[...]


def _b_body(S_ref, h_ref, dinv_ref, c1a_ref, wc_ref, wm_ref,
            whz_ref, bhz_ref, whr_ref, bhr_ref,
            y2_ref, y2s_ref, hz_ref, hr_ref):
    dinv = dinv_ref[...]
    h = h_ref[...]
    ah = dinv * (S_ref[0] + S_ref[1]) + (dinv * dinv) * h
    hc = jnp.maximum(c1a_ref[...] + _mm(ah, wc_ref[...]), 0.0)
    y2 = _mm(hc, wm_ref[...])
    y2_ref[...] = y2
    y2s_ref[...] = y2 * dinv
    hz_ref[...] = _mm(ah, whz_ref[...]) + bhz_ref[...][None, :]
    hr_ref[...] = _mm(ah, whr_ref[...]) + bhr_ref[...][None, :]


def _c_body(S_ref, y2_ref, dinv_ref, bm_ref, wpz_ref, bpz_ref,
            phi_ref, phis_ref):
    dinv = dinv_ref[...]
    z_enc = dinv * (S_ref[0] + S_ref[1]) \
        + (dinv * dinv) * y2_ref[...] + bm_ref[...][None, :]
    phi = jnp.maximum(_mm(z_enc, wpz_ref[...]) + bpz_ref[...][None, :], 0.0)
    phi_ref[...] = phi
    phis_ref[...] = phi * dinv


def _d0a_body(S_ref, phi_ref, dinv_ref, xza_ref, xha_ref,
              wzb_ref, whb_ref, bhz_ref, bhh_ref,
              wpr_ref, bpr_ref, wpm_ref, bpm_ref,
              h_ref, z_ref, hs_ref):
    dinv = dinv_ref[...]
    aphi = dinv * (S_ref[0] + S_ref[1]) + (dinv * dinv) * phi_ref[...]
    z_g = jax.nn.sigmoid(xza_ref[...] + _mm(aphi, wzb_ref[...])
                         + bhz_ref[...][None, :])
    xh = xha_ref[...] + _mm(aphi, whb_ref[...])
    h_hat = jnp.tanh(xh + bhh_ref[...][None, :])
    h = (1.0 - z_g) * h_hat
    h_ref[...] = h
    prior = jnp.maximum(_mm(h, wpr_ref[...]) + bpr_ref[...][None, :], 0.0)
    z_ref[...] = _mm(prior, wpm_ref[...]) + bpm_ref[...][None, :]
    hs_ref[...] = h * dinv


def _d_body(S_ref, phi_ref, dinv_ref, xza_ref, xra_ref, xha_ref,
            wzb_ref, wrb_ref, whb_ref, hz_ref, hr_ref, h_ref, whh_ref,
            zg_ref, xh_ref, y4_ref, y4s_ref):
    dinv = dinv_ref[...]
    aphi = dinv * (S_ref[0] + S_ref[1]) + (dinv * dinv) * phi_ref[...]
    z_g = jax.nn.sigmoid(xza_ref[...] + _mm(aphi, wzb_ref[...]) + hz_ref[...])
    r_g = jax.nn.sigmoid(xra_ref[...] + _mm(aphi, wrb_ref[...]) + hr_ref[...])
    zg_ref[...] = z_g
    xh_ref[...] = xha_ref[...] + _mm(aphi, whb_ref[...])
    y4 = _mm(r_g * h_ref[...], whh_ref[...])
    y4_ref[...] = y4
    y4s_ref[...] = y4 * dinv


def _ea_body(S_ref, y4_ref, dinv_ref, bhh_ref, xh_ref, zg_ref, hold_ref,
             wpr_ref, bpr_ref, wpm_ref, bpm_ref,
             h_ref, z_ref, hs_ref):
    dinv = dinv_ref[...]
    hh = dinv * (S_ref[0] + S_ref[1]) \
        + (dinv * dinv) * y4_ref[...] + bhh_ref[...][None, :]
    h_hat = jnp.tanh(xh_ref[...] + hh)
    z_g = zg_ref[...]
    h = z_g * hold_ref[0:1, :] + (1.0 - z_g) * h_hat
    h_ref[...] = h
    prior = jnp.maximum(_mm(h, wpr_ref[...]) + bpr_ref[...][None, :], 0.0)
    z_ref[...] = _mm(prior, wpm_ref[...]) + bpm_ref[...][None, :]
    hs_ref[...] = h * dinv


def _e_body(S_ref, y4_ref, dinv_ref, bhh_ref, xh_ref, zg_ref, hold_ref,
            h_ref):
    dinv = dinv_ref[...]
    hh = dinv * (S_ref[0] + S_ref[1]) \
        + (dinv * dinv) * y4_ref[...] + bhh_ref[...][None, :]
    h_hat = jnp.tanh(xh_ref[...] + hh)
    z_g = zg_ref[...]
    h_ref[...] = z_g * hold_ref[0:1, :] + (1.0 - z_g) * h_hat


_f32 = jnp.float32


def _sds(*shape):
    return jax.ShapeDtypeStruct(shape, _f32)


def kernel(xs, edge_index, W_phi_x, b_phi_x, W_prior, b_prior, W_pmean,
           b_pmean, W_pstd, b_pstd, W_c1, b_c1, W_mean, b_mean, W_phi_z,
           b_phi_z, W_xz, b_xz, W_hz, b_hz, W_xr, b_xr, W_hr, b_hr, W_xh,
           b_xh, W_hh, b_hh):
    s = edge_index[0]
    d = edge_index[1]
    pad = EPAD - E
    s3 = jnp.concatenate([s, jnp.zeros((pad,), jnp.int32)]) \
        .reshape(NTILES, CPT, CHUNK)
    d3 = jnp.concatenate([d, jnp.full((pad,), N, jnp.int32)]) \
        .reshape(NTILES, CPT, CHUNK)

    W_c1a, W_c1b = W_c1[:HD], W_c1[HD:]
    W_xza, W_xzb = W_xz[:HD], W_xz[HD:]
    W_xra, W_xrb = W_xr[:HD], W_xr[HD:]
    W_xha, W_xhb = W_xh[:HD], W_xh[HD:]

    w64 = _whole((HD, HD))
    w6432 = _whole((HD, ZD))
    w3264 = _whole((ZD, HD))
    b64 = _whole((HD,))
    b32 = _whole((ZD,))

    degp = _sc_deg(d3).reshape(NCORES, NP, 1)
    dinv, xp, xps = _tc(
        _prep_body, [_sds(N, 1), _sds(N, HD), _sds(N, HD)],
        [_Ssp(1), _row(XD), _whole((XD, HD)), b64],
        [_row(1), _row(HD), _row(HD)],
        degp, xs, W_phi_x, b_phi_x)

    Sxp = _sc_scatter64(xps, s3, d3)
    C1a, XZa, XRa, XHa, z0 = _tc(
        _axp_body,
        [_sds(N, HD), _sds(N, HD), _sds(N, HD), _sds(N, HD), _sds(N, ZD)],
        [_Ssp(HD), _row(HD), _row(1), w64, b64, w64, b64, w64, b64,
         w64, b64, b64, w6432, b32],
        [_row(HD), _row(HD), _row(HD), _row(HD), _row(ZD)],
        Sxp, xp, dinv, W_c1a, b_c1, W_xza, b_xz, W_xra, b_xr, W_xha, b_xh,
        b_prior, W_pmean, b_pmean)

    y2, y2s = _tc(
        _b0_body, [_sds(N, ZD), _sds(N, ZD)],
        [_row(HD), _row(1), w6432],
        [_row(ZD), _row(ZD)],
        C1a, dinv, W_mean)
    Sz = _sc_scatter32(y2s, s3, d3)
    phi, phis = _tc(
        _c_body, [_sds(N, HD), _sds(N, HD)],
        [_Ssp(ZD), _row(ZD), _row(1), b32, w3264, b64],
        [_row(HD), _row(HD)],
        Sz, y2, dinv, b_mean, W_phi_z, b_phi_z)
    Sphi = _sc_scatter64(phis, s3, d3)
    h, z1, hs = _tc(
        _d0a_body, [_sds(N, HD), _sds(N, ZD), _sds(N, HD)],
        [_Ssp(HD), _row(HD), _row(1), _row(HD), _row(HD),
         w64, w64, b64, b64, w64, b64, w6432, b32],
        [_row(HD), _row(ZD), _row(HD)],
        Sphi, phi, dinv, XZa, XHa, W_xzb, W_xhb, b_hz, b_hh,
        W_prior, b_prior, W_pmean, b_pmean)

    zs = [z0, z1]
    for t in (1, 2):
        Sh = _sc_scatter64(hs, s3, d3)
        y2, y2s, hz, hr = _tc(
            _b_body, [_sds(N, ZD), _sds(N, ZD), _sds(N, HD), _sds(N, HD)],
            [_Ssp(HD), _row(HD), _row(1), _row(HD), w64, w6432,
             w64, b64, w64, b64],
            [_row(ZD), _row(ZD), _row(HD), _row(HD)],
            Sh, h, dinv, C1a, W_c1b, W_mean, W_hz, b_hz, W_hr, b_hr)
        Sz = _sc_scatter32(y2s, s3, d3)
        phi, phis = _tc(
            _c_body, [_sds(N, HD), _sds(N, HD)],
            [_Ssp(ZD), _row(ZD), _row(1), b32, w3264, b64],
            [_row(HD), _row(HD)],
            Sz, y2, dinv, b_mean, W_phi_z, b_phi_z)
        Sphi = _sc_scatter64(phis, s3, d3)
        zg, xh, y4, y4s = _tc(
            _d_body, [_sds(N, HD), _sds(N, HD), _sds(N, HD), _sds(N, HD)],
            [_Ssp(HD), _row(HD), _row(1), _row(HD), _row(HD), _row(HD),
             w64, w64, w64, _row(HD), _row(HD), _row(HD), w64],
            [_row(HD), _row(HD), _row(HD), _row(HD)],
            Sphi, phi, dinv, XZa, XRa, XHa, W_xzb, W_xrb, W_xhb,
            hz, hr, h, W_hh)
        Su = _sc_scatter64(y4s, s3, d3)
        if t < 2:
            h, z_t, hs = _tc(
                _ea_body, [_sds(N, HD), _sds(N, ZD), _sds(N, HD)],
                [_Ssp(HD), _row(HD), _row(1), b64, _row(HD), _row(HD),
                 _row0(HD), w64, b64, w6432, b32],
                [_row(HD), _row(ZD), _row(HD)],
                Su, y4, dinv, b_hh, xh, zg, h,
                W_prior, b_prior, W_pmean, b_pmean)
            zs.append(z_t)
        else:
            h = _tc(
                _e_body, _sds(N, HD),
                [_Ssp(HD), _row(HD), _row(1), b64, _row(HD), _row(HD),
                 _row0(HD)],
                _row(HD),
                Su, y4, dinv, b_hh, xh, zg, h)

    return jnp.stack(zs), h


# speedup vs baseline: 2.1972x; 2.0360x over previous
import functools

import jax
import jax.numpy as jnp
from jax import lax
from jax.experimental import pallas as pl
from jax.experimental.pallas import tpu as pltpu
from jax.experimental.pallas import tpu_sc as plsc

N = 10000
XD = 128
HD = 64
ZD = 32
E = 320000

NCORES = 2
NSUB = 16
NTILES = NCORES * NSUB
CHUNK = 128
CPT = 80
EPAD = NTILES * CPT * CHUNK
NP = 10240
RPT = NP // NSUB
NBUF = 2
_DO_GATHER = True
_DO_SCATTER = True
XRPT = N // NSUB

TB = 2000
GRID = N // TB


def _sc_scatter_body(W, x_hbm, s_hbm, d_hbm, out_hbm,
                     s_v, d_v, *rest):
    c = lax.axis_index("c")
    t = lax.axis_index("s")
    wid = t * NCORES + c
    bufs = rest[:NBUF]
    acc = rest[NBUF]
    xsp = rest[NBUF + 1]
    gsems = rest[NBUF + 2:2 * NBUF + 2]
    ssems = rest[2 * NBUF + 2:3 * NBUF + 2]
    semz = rest[3 * NBUF + 2]
    b0 = bufs[0]

    z16 = jnp.zeros((16,), jnp.float32)

    def _zb(i, carry):
        for j in range(W // 16):
            b0[i, pl.ds(j * 16, 16)] = z16
        return carry

    lax.fori_loop(0, CHUNK, _zb, 0)
    zh = []
    for kk in range(RPT // CHUNK):
        zh.append(pltpu.async_copy(
            b0, acc.at[pl.ds(t * RPT + kk * CHUNK, CHUNK)], semz))
    for h in zh:
        h.wait()
    pltpu.sync_copy(x_hbm.at[pl.ds(t * XRPT, XRPT)],
                    xsp.at[pl.ds(t * XRPT, XRPT)])
    plsc.subcore_barrier()

    pltpu.sync_copy(s_hbm.at[wid], s_v)
    pltpu.sync_copy(d_hbm.at[wid], d_v)

    gh0 = []
    for b in range(NBUF):
        gh0.append(pltpu.async_copy(xsp.at[s_v.at[b]], bufs[b], gsems[b]))
    for b in range(NBUF):
        gh0[b].wait()
        if _DO_SCATTER:
            pltpu.async_copy(bufs[b], acc.at[d_v.at[b]], ssems[b], add=True)

    def _grp(g, carry):
        base = g * NBUF
        ghs = []
        for b in range(NBUF):
            if _DO_SCATTER:
                pltpu.make_async_copy(
                    bufs[b], acc.at[d_v.at[base + b]], ssems[b]).wait()
            if _DO_GATHER:
                ghs.append(pltpu.async_copy(
                    xsp.at[s_v.at[base + b]], bufs[b], gsems[b]))
        for b in range(NBUF):
            if _DO_GATHER:
                ghs[b].wait()
            if _DO_SCATTER:
                pltpu.async_copy(
                    bufs[b], acc.at[d_v.at[base + b]], ssems[b], add=True)
        return carry

    lax.fori_loop(1, CPT // NBUF, _grp, 0)
    if _DO_SCATTER:
        for b in range(NBUF):
            pltpu.make_async_copy(bufs[b], acc.at[d_v.at[b]], ssems[b]).wait()
    plsc.subcore_barrier()

    pltpu.sync_copy(acc.at[pl.ds(t * RPT, RPT)],
                    out_hbm.at[c, pl.ds(t * RPT, RPT)])


@functools.lru_cache(maxsize=None)
def _make_sc_scatter(W):
    body = functools.partial(_sc_scatter_body, W)
    mesh = plsc.VectorSubcoreMesh(core_axis_name="c", subcore_axis_name="s")
    return pl.kernel(
        body,
        out_type=jax.ShapeDtypeStruct((NCORES, NP, W), jnp.float32),
        mesh=mesh,
        compiler_params=pltpu.CompilerParams(use_tc_tiling_on_sc=False),
        scratch_types=(
            [pltpu.VMEM((CPT, CHUNK), jnp.int32),
             pltpu.VMEM((CPT, CHUNK), jnp.int32)]
            + [pltpu.VMEM((CHUNK, W), jnp.float32)] * NBUF
            + [pltpu.VMEM_SHARED((NP, W), jnp.float32)]
            + [pltpu.VMEM_SHARED((N, W), jnp.float32)]
            + [pltpu.SemaphoreType.DMA] * (2 * NBUF + 1)
        ),
    )


def _sc_deg_body(d_hbm, out_hbm, d_v, ones_v, zb, acc, semz):
    c = lax.axis_index("c")
    t = lax.axis_index("s")
    wid = t * NCORES + c

    one16 = jnp.ones((16,), jnp.float32)
    z16 = jnp.zeros((16,), jnp.float32)

    def _fill(i, carry):
        ones_v[pl.ds(i * 16, 16)] = one16
        return carry

    lax.fori_loop(0, CHUNK // 16, _fill, 0)

    def _zb(i, carry):
        zb[pl.ds(i * 16, 16)] = z16
        return carry

    lax.fori_loop(0, RPT // 16, _zb, 0)
    pltpu.async_copy(zb, acc.at[pl.ds(t * RPT, RPT)], semz).wait()
    plsc.subcore_barrier()

    pltpu.sync_copy(d_hbm.at[wid], d_v)

    def _step(j, carry):
        pltpu.async_copy(ones_v, acc.at[d_v.at[j]], semz, add=True)
        return carry

    lax.fori_loop(0, CPT, _step, 0)

    def _drain(j, carry):
        pltpu.make_async_copy(ones_v, acc.at[d_v.at[j]], semz).wait()
        return carry

    lax.fori_loop(0, CPT, _drain, 0)
    plsc.subcore_barrier()

    pltpu.sync_copy(acc.at[pl.ds(t * RPT, RPT)],
                    out_hbm.at[c, pl.ds(t * RPT, RPT)])


@functools.lru_cache(maxsize=None)
def _make_sc_deg():
    mesh = plsc.VectorSubcoreMesh(core_axis_name="c", subcore_axis_name="s")
    return pl.kernel(
        _sc_deg_body,
        out_type=jax.ShapeDtypeStruct((NCORES, NP), jnp.float32),
        mesh=mesh,
        compiler_params=pltpu.CompilerParams(use_tc_tiling_on_sc=False),
        scratch_types=[
            pltpu.VMEM((CPT, CHUNK), jnp.int32),
            pltpu.VMEM((CHUNK,), jnp.float32),
            pltpu.VMEM((RPT,), jnp.float32),
            pltpu.VMEM_SHARED((NP,), jnp.float32),
            pltpu.SemaphoreType.DMA,
        ],
    )


def _sc_deg(d3):
    return _make_sc_deg()(d3)


def _sc_scatter64(x, s3, d3):
    return _make_sc_scatter(HD)(x, s3, d3)


def _sc_scatter32(x, s3, d3):
    return _make_sc_scatter(ZD)(x, s3, d3)


def _mm(a, w):
    return jnp.dot(a, w, preferred_element_type=jnp.float32)


def _row(w):
    return pl.BlockSpec((TB, w), lambda i: (i, 0))


def _Ssp(w):
    return pl.BlockSpec((NCORES, TB, w), lambda i: (0, i, 0))


def _whole(shape):
    return pl.BlockSpec(shape, lambda i, _n=len(shape): (0,) * _n)


def _row0(w):
    return pl.BlockSpec((8, w), lambda i: (0, 0))


def _tc(body, out_shapes, in_specs, out_specs, *args):
    return pl.pallas_call(
        body, out_shape=out_shapes, grid=(GRID,),
        in_specs=in_specs, out_specs=out_specs,
        compiler_params=pltpu.CompilerParams(
            dimension_semantics=("arbitrary",)),
    )(*args)


def _prep_body(degp_ref, xs_ref, w_ref, b_ref, dinv_ref, xp_ref, xps_ref):
    deg = degp_ref[0] + degp_ref[1] + 1.0
    dinv = lax.rsqrt(deg)
    dinv_ref[...] = dinv
    xp = jnp.maximum(_mm(xs_ref[...], w_ref[...]) + b_ref[...][None, :], 0.0)
    xp_ref[...] = xp
    xps_ref[...] = xp * dinv


def _axp_body(S_ref, xp_ref, dinv_ref, wc_ref, bc_ref, wz_ref, bz_ref,
              wr_ref, br_ref, wh_ref, bh_ref, bp_ref, wpm_ref, bpm_ref,
              c1a_ref, xza_ref, xra_ref, xha_ref, z0_ref):
    dinv = dinv_ref[...]
    axp = dinv * (S_ref[0] + S_ref[1]) + (dinv * dinv) * xp_ref[...]
    c1a_ref[...] = _mm(axp, wc_ref[...]) + bc_ref[...][None, :]
    xza_ref[...] = _mm(axp, wz_ref[...]) + bz_ref[...][None, :]
    xra_ref[...] = _mm(axp, wr_ref[...]) + br_ref[...][None, :]
    xha_ref[...] = _mm(axp, wh_ref[...]) + bh_ref[...][None, :]
    pr0 = jnp.maximum(bp_ref[...], 0.0)[None, :]
    z0row = _mm(pr0, wpm_ref[...]) + bpm_ref[...][None, :]
    z0_ref[...] = jnp.broadcast_to(z0row, (TB, ZD))


def _b0_body(c1a_ref, dinv_ref, wm_ref, y2_ref, y2s_ref):
    hc = jnp.maximum(c1a_ref[...], 0.0)
    y2 = _mm(hc, wm_ref[...])
    y2_ref[...] = y2
    y2s_ref[...] = y2 * dinv_ref[...]


def _b_body(S_ref, h_ref, dinv_ref, c1a_ref, wc_ref, wm_ref,
            whz_ref, bhz_ref, whr_ref, bhr_ref,
            y2_ref, y2s_ref, hz_ref, hr_ref):
    dinv = dinv_ref[...]
    h = h_ref[...]
    ah = dinv * (S_ref[0] + S_ref[1]) + (dinv * dinv) * h
    hc = jnp.maximum(c1a_ref[...] + _mm(ah, wc_ref[...]), 0.0)
    y2 = _mm(hc, wm_ref[...])
    y2_ref[...] = y2
    y2s_ref[...] = y2 * dinv
    hz_ref[...] = _mm(ah, whz_ref[...]) + bhz_ref[...][None, :]
    hr_ref[...] = _mm(ah, whr_ref[...]) + bhr_ref[...][None, :]


def _c_body(S_ref, y2_ref, dinv_ref, bm_ref, wpz_ref, bpz_ref,
            phi_ref, phis_ref):
    dinv = dinv_ref[...]
    z_enc = dinv * (S_ref[0] + S_ref[1]) \
        + (dinv * dinv) * y2_ref[...] + bm_ref[...][None, :]
    phi = jnp.maximum(_mm(z_enc, wpz_ref[...]) + bpz_ref[...][None, :], 0.0)
    phi_ref[...] = phi
    phis_ref[...] = phi * dinv


def _d0a_body(S_ref, phi_ref, dinv_ref, xza_ref, xha_ref,
              wzb_ref, whb_ref, bhz_ref, bhh_ref,
              wpr_ref, bpr_ref, wpm_ref, bpm_ref,
              h_ref, z_ref, hs_ref):
    dinv = dinv_ref[...]
    aphi = dinv * (S_ref[0] + S_ref[1]) + (dinv * dinv) * phi_ref[...]
    z_g = jax.nn.sigmoid(xza_ref[...] + _mm(aphi, wzb_ref[...])
                         + bhz_ref[...][None, :])
    xh = xha_ref[...] + _mm(aphi, whb_ref[...])
    h_hat = jnp.tanh(xh + bhh_ref[...][None, :])
    h = (1.0 - z_g) * h_hat
    h_ref[...] = h
    prior = jnp.maximum(_mm(h, wpr_ref[...]) + bpr_ref[...][None, :], 0.0)
    z_ref[...] = _mm(prior, wpm_ref[...]) + bpm_ref[...][None, :]
    hs_ref[...] = h * dinv


def _d_body(S_ref, phi_ref, dinv_ref, xza_ref, xra_ref, xha_ref,
            wzb_ref, wrb_ref, whb_ref, hz_ref, hr_ref, h_ref, whh_ref,
            zg_ref, xh_ref, y4_ref, y4s_ref):
    dinv = dinv_ref[...]
    aphi = dinv * (S_ref[0] + S_ref[1]) + (dinv * dinv) * phi_ref[...]
    z_g = jax.nn.sigmoid(xza_ref[...] + _mm(aphi, wzb_ref[...]) + hz_ref[...])
    r_g = jax.nn.sigmoid(xra_ref[...] + _mm(aphi, wrb_ref[...]) + hr_ref[...])
    zg_ref[...] = z_g
    xh_ref[...] = xha_ref[...] + _mm(aphi, whb_ref[...])
    y4 = _mm(r_g * h_ref[...], whh_ref[...])
    y4_ref[...] = y4
    y4s_ref[...] = y4 * dinv


def _ea_body(S_ref, y4_ref, dinv_ref, bhh_ref, xh_ref, zg_ref, hold_ref,
             wpr_ref, bpr_ref, wpm_ref, bpm_ref,
             h_ref, z_ref, hs_ref):
    dinv = dinv_ref[...]
    hh = dinv * (S_ref[0] + S_ref[1]) \
        + (dinv * dinv) * y4_ref[...] + bhh_ref[...][None, :]
    h_hat = jnp.tanh(xh_ref[...] + hh)
    z_g = zg_ref[...]
    h = z_g * hold_ref[0:1, :] + (1.0 - z_g) * h_hat
    h_ref[...] = h
    prior = jnp.maximum(_mm(h, wpr_ref[...]) + bpr_ref[...][None, :], 0.0)
    z_ref[...] = _mm(prior, wpm_ref[...]) + bpm_ref[...][None, :]
    hs_ref[...] = h * dinv


def _e_body(S_ref, y4_ref, dinv_ref, bhh_ref, xh_ref, zg_ref, hold_ref,
            h_ref):
    dinv = dinv_ref[...]
    hh = dinv * (S_ref[0] + S_ref[1]) \
        + (dinv * dinv) * y4_ref[...] + bhh_ref[...][None, :]
    h_hat = jnp.tanh(xh_ref[...] + hh)
    z_g = zg_ref[...]
    h_ref[...] = z_g * hold_ref[0:1, :] + (1.0 - z_g) * h_hat


_f32 = jnp.float32


def _sds(*shape):
    return jax.ShapeDtypeStruct(shape, _f32)


def kernel(xs, edge_index, W_phi_x, b_phi_x, W_prior, b_prior, W_pmean,
           b_pmean, W_pstd, b_pstd, W_c1, b_c1, W_mean, b_mean, W_phi_z,
           b_phi_z, W_xz, b_xz, W_hz, b_hz, W_xr, b_xr, W_hr, b_hr, W_xh,
           b_xh, W_hh, b_hh):
    s = edge_index[0]
    d = edge_index[1]
    pad = EPAD - E
    s3 = jnp.concatenate([s, jnp.zeros((pad,), jnp.int32)]) \
        .reshape(NTILES, CPT, CHUNK)
    d3 = jnp.concatenate([d, jnp.full((pad,), N, jnp.int32)]) \
        .reshape(NTILES, CPT, CHUNK)

    W_c1a, W_c1b = W_c1[:HD], W_c1[HD:]
    W_xza, W_xzb = W_xz[:HD], W_xz[HD:]
    W_xra, W_xrb = W_xr[:HD], W_xr[HD:]
    W_xha, W_xhb = W_xh[:HD], W_xh[HD:]

    w64 = _whole((HD, HD))
    w6432 = _whole((HD, ZD))
    w3264 = _whole((ZD, HD))
    b64 = _whole((HD,))
    b32 = _whole((ZD,))

    degp = _sc_deg(d3).reshape(NCORES, NP, 1)
    dinv, xp, xps = _tc(
        _prep_body, [_sds(N, 1), _sds(N, HD), _sds(N, HD)],
        [_Ssp(1), _row(XD), _whole((XD, HD)), b64],
        [_row(1), _row(HD), _row(HD)],
        degp, xs, W_phi_x, b_phi_x)

    Sxp = _sc_scatter64(xps, s3, d3)
    C1a, XZa, XRa, XHa, z0 = _tc(
        _axp_body,
        [_sds(N, HD), _sds(N, HD), _sds(N, HD), _sds(N, HD), _sds(N, ZD)],
        [_Ssp(HD), _row(HD), _row(1), w64, b64, w64, b64, w64, b64,
         w64, b64, b64, w6432, b32],
        [_row(HD), _row(HD), _row(HD), _row(HD), _row(ZD)],
        Sxp, xp, dinv, W_c1a, b_c1, W_xza, b_xz, W_xra, b_xr, W_xha, b_xh,
        b_prior, W_pmean, b_pmean)

    y2, y2s = _tc(
        _b0_body, [_sds(N, ZD), _sds(N, ZD)],
        [_row(HD), _row(1), w6432],
        [_row(ZD), _row(ZD)],
        C1a, dinv, W_mean)
    Sz = _sc_scatter32(y2s, s3, d3)
    phi, phis = _tc(
        _c_body, [_sds(N, HD), _sds(N, HD)],
        [_Ssp(ZD), _row(ZD), _row(1), b32, w3264, b64],
        [_row(HD), _row(HD)],
        Sz, y2, dinv, b_mean, W_phi_z, b_phi_z)
    Sphi = _sc_scatter64(phis, s3, d3)
    h, z1, hs = _tc(
        _d0a_body, [_sds(N, HD), _sds(N, ZD), _sds(N, HD)],
        [_Ssp(HD), _row(HD), _row(1), _row(HD), _row(HD),
         w64, w64, b64, b64, w64, b64, w6432, b32],
        [_row(HD), _row(ZD), _row(HD)],
        Sphi, phi, dinv, XZa, XHa, W_xzb, W_xhb, b_hz, b_hh,
        W_prior, b_prior, W_pmean, b_pmean)

    zs = [z0, z1]
    for t in (1, 2):
        Sh = _sc_scatter64(hs, s3, d3)
        y2, y2s, hz, hr = _tc(
            _b_body, [_sds(N, ZD), _sds(N, ZD), _sds(N, HD), _sds(N, HD)],
            [_Ssp(HD), _row(HD), _row(1), _row(HD), w64, w6432,
             w64, b64, w64, b64],
            [_row(ZD), _row(ZD), _row(HD), _row(HD)],
            Sh, h, dinv, C1a, W_c1b, W_mean, W_hz, b_hz, W_hr, b_hr)
        Sz = _sc_scatter32(y2s, s3, d3)
        phi, phis = _tc(
            _c_body, [_sds(N, HD), _sds(N, HD)],
            [_Ssp(ZD), _row(ZD), _row(1), b32, w3264, b64],
            [_row(HD), _row(HD)],
            Sz, y2, dinv, b_mean, W_phi_z, b_phi_z)
        Sphi = _sc_scatter64(phis, s3, d3)
        zg, xh, y4, y4s = _tc(
            _d_body, [_sds(N, HD), _sds(N, HD), _sds(N, HD), _sds(N, HD)],
            [_Ssp(HD), _row(HD), _row(1), _row(HD), _row(HD), _row(HD),
             w64, w64, w64, _row(HD), _row(HD), _row(HD), w64],
            [_row(HD), _row(HD), _row(HD), _row(HD)],
            Sphi, phi, dinv, XZa, XRa, XHa, W_xzb, W_xrb, W_xhb,
            hz, hr, h, W_hh)
        Su = _sc_scatter64(y4s, s3, d3)
        if t < 2:
            h, z_t, hs = _tc(
                _ea_body, [_sds(N, HD), _sds(N, ZD), _sds(N, HD)],
                [_Ssp(HD), _row(HD), _row(1), b64, _row(HD), _row(HD),
                 _row0(HD), w64, b64, w6432, b32],
                [_row(HD), _row(ZD), _row(HD)],
                Su, y4, dinv, b_hh, xh, zg, h,
                W_prior, b_prior, W_pmean, b_pmean)
            zs.append(z_t)
        else:
            h = _tc(
                _e_body, _sds(N, HD),
                [_Ssp(HD), _row(HD), _row(1), b64, _row(HD), _row(HD),
                 _row0(HD)],
                _row(HD),
                Su, y4, dinv, b_hh, xh, zg, h)

    return jnp.stack(zs), h


# speedup vs baseline: 2.2052x; 1.0037x over previous
import functools

import jax
import jax.numpy as jnp
from jax import lax
from jax.experimental import pallas as pl
from jax.experimental.pallas import tpu as pltpu
from jax.experimental.pallas import tpu_sc as plsc

N = 10000
XD = 128
HD = 64
ZD = 32
E = 320000

NCORES = 2
NSUB = 16
NTILES = NCORES * NSUB
CHUNK = 128
CPT = 80
EPAD = NTILES * CPT * CHUNK
NP = 10240
RPT = NP // NSUB
XRPT = N // NSUB

TB = 2000
GRID = N // TB


def _sc_scatter_body(W, NB, x_hbm, s_hbm, d_hbm, out_hbm,
                     s_g, d_g, *rest):
    c = lax.axis_index("c")
    t = lax.axis_index("s")
    wid = t * NCORES + c
    bufs = rest[:NB]
    acc = rest[NB]
    xsp = rest[NB + 1]
    gsems = rest[NB + 2:2 * NB + 2]
    ssems = rest[2 * NB + 2:3 * NB + 2]
    semz = rest[3 * NB + 2]
    xsem = rest[3 * NB + 3]
    isem = rest[3 * NB + 4]
    b0 = bufs[0]

    z16 = jnp.zeros((16,), jnp.float32)

    def _zb(i, carry):
        for j in range(W // 16):
            b0[i, pl.ds(j * 16, 16)] = z16
        return carry

    lax.fori_loop(0, CHUNK, _zb, 0)
    zh = []
    for kk in range(RPT // CHUNK):
        zh.append(pltpu.async_copy(
            b0, acc.at[pl.ds(t * RPT + kk * CHUNK, CHUNK)], semz))
    xh = pltpu.async_copy(x_hbm.at[pl.ds(t * XRPT, XRPT)],
                          xsp.at[pl.ds(t * XRPT, XRPT)], xsem)
    pltpu.sync_copy(s_hbm.at[wid, pl.ds(0, NB)], s_g.at[0])
    pltpu.sync_copy(d_hbm.at[wid, pl.ds(0, NB)], d_g.at[0])
    for h in zh:
        h.wait()
    xh.wait()
    plsc.subcore_barrier()

    NG = CPT // NB
    pend_scat = [None] * NB
    pend_idx = None
    for g in range(NG):
        slot = g % 2
        for b in range(NB):
            if pend_scat[b] is not None:
                pend_scat[b].wait()
        if g + 1 < NG:
            nslot = (g + 1) % 2
            pend_idx_next = (
                pltpu.async_copy(s_hbm.at[wid, pl.ds((g + 1) * NB, NB)],
                                 s_g.at[nslot], isem),
                pltpu.async_copy(d_hbm.at[wid, pl.ds((g + 1) * NB, NB)],
                                 d_g.at[nslot], isem))
        else:
            pend_idx_next = None
        if pend_idx is not None:
            pend_idx[0].wait()
            pend_idx[1].wait()
        ghs = []
        for b in range(NB):
            ghs.append(pltpu.async_copy(
                xsp.at[s_g.at[slot, b]], bufs[b], gsems[b]))
        for b in range(NB):
            ghs[b].wait()
            pend_scat[b] = pltpu.async_copy(
                bufs[b], acc.at[d_g.at[slot, b]], ssems[b], add=True)
        pend_idx = pend_idx_next
    for b in range(NB):
        pend_scat[b].wait()
    plsc.subcore_barrier()

    pltpu.sync_copy(acc.at[pl.ds(t * RPT, RPT)],
                    out_hbm.at[c, pl.ds(t * RPT, RPT)])


@functools.lru_cache(maxsize=None)
def _make_sc_scatter(W, NB):
    body = functools.partial(_sc_scatter_body, W, NB)
    mesh = plsc.VectorSubcoreMesh(core_axis_name="c", subcore_axis_name="s")
    return pl.kernel(
        body,
        out_type=jax.ShapeDtypeStruct((NCORES, NP, W), jnp.float32),
        mesh=mesh,
        compiler_params=pltpu.CompilerParams(use_tc_tiling_on_sc=False),
        scratch_types=(
            [pltpu.VMEM((2, NB, CHUNK), jnp.int32),
             pltpu.VMEM((2, NB, CHUNK), jnp.int32)]
            + [pltpu.VMEM((CHUNK, W), jnp.float32)] * NB
            + [pltpu.VMEM_SHARED((NP, W), jnp.float32)]
            + [pltpu.VMEM_SHARED((N, W), jnp.float32)]
            + [pltpu.SemaphoreType.DMA] * (2 * NB + 3)
        ),
    )


def _sc_deg_body(d_hbm, out_hbm, d_v, ones_v, zb, acc, semz):
    c = lax.axis_index("c")
    t = lax.axis_index("s")
    wid = t * NCORES + c

    one16 = jnp.ones((16,), jnp.float32)
    z16 = jnp.zeros((16,), jnp.float32)

    def _fill(i, carry):
        ones_v[pl.ds(i * 16, 16)] = one16
        return carry

    lax.fori_loop(0, CHUNK // 16, _fill, 0)

    def _zb(i, carry):
        zb[pl.ds(i * 16, 16)] = z16
        return carry

    lax.fori_loop(0, RPT // 16, _zb, 0)
    pltpu.async_copy(zb, acc.at[pl.ds(t * RPT, RPT)], semz).wait()
    plsc.subcore_barrier()

    pltpu.sync_copy(d_hbm.at[wid], d_v)

    def _step(j, carry):
        pltpu.async_copy(ones_v, acc.at[d_v.at[j]], semz, add=True)
        return carry

    lax.fori_loop(0, CPT, _step, 0)

    def _drain(j, carry):
        pltpu.make_async_copy(ones_v, acc.at[d_v.at[j]], semz).wait()
        return carry

    lax.fori_loop(0, CPT, _drain, 0)
    plsc.subcore_barrier()

    pltpu.sync_copy(acc.at[pl.ds(t * RPT, RPT)],
                    out_hbm.at[c, pl.ds(t * RPT, RPT)])


@functools.lru_cache(maxsize=None)
def _make_sc_deg():
    mesh = plsc.VectorSubcoreMesh(core_axis_name="c", subcore_axis_name="s")
    return pl.kernel(
        _sc_deg_body,
        out_type=jax.ShapeDtypeStruct((NCORES, NP), jnp.float32),
        mesh=mesh,
        compiler_params=pltpu.CompilerParams(use_tc_tiling_on_sc=False),
        scratch_types=[
            pltpu.VMEM((CPT, CHUNK), jnp.int32),
            pltpu.VMEM((CHUNK,), jnp.float32),
            pltpu.VMEM((RPT,), jnp.float32),
            pltpu.VMEM_SHARED((NP,), jnp.float32),
            pltpu.SemaphoreType.DMA,
        ],
    )


def _sc_deg(d3):
    return _make_sc_deg()(d3)


def _sc_scatter64(x, s3, d3):
    return _make_sc_scatter(HD, 4)(x, s3, d3)


def _sc_scatter32(x, s3, d3):
    return _make_sc_scatter(ZD, 8)(x, s3, d3)


def _mm(a, w):
    return jnp.dot(a, w, preferred_element_type=jnp.float32)


def _row(w):
    return pl.BlockSpec((TB, w), lambda i: (i, 0))


def _Ssp(w):
    return pl.BlockSpec((NCORES, TB, w), lambda i: (0, i, 0))


def _whole(shape):
    return pl.BlockSpec(shape, lambda i, _n=len(shape): (0,) * _n)


def _row0(w):
    return pl.BlockSpec((8, w), lambda i: (0, 0))


def _tc(body, out_shapes, in_specs, out_specs, *args):
    return pl.pallas_call(
        body, out_shape=out_shapes, grid=(GRID,),
        in_specs=in_specs, out_specs=out_specs,
        compiler_params=pltpu.CompilerParams(
            dimension_semantics=("arbitrary",)),
    )(*args)


def _prep_body(degp_ref, xs_ref, w_ref, b_ref, dinv_ref, xp_ref, xps_ref):
    deg = degp_ref[0] + degp_ref[1] + 1.0
    dinv = lax.rsqrt(deg)
    dinv_ref[...] = dinv
    xp = jnp.maximum(_mm(xs_ref[...], w_ref[...]) + b_ref[...][None, :], 0.0)
    xp_ref[...] = xp
    xps_ref[...] = xp * dinv


def _axp_body(S_ref, xp_ref, dinv_ref, wc_ref, bc_ref, wz_ref, bz_ref,
              wr_ref, br_ref, wh_ref, bh_ref, bp_ref, wpm_ref, bpm_ref,
              c1a_ref, xza_ref, xra_ref, xha_ref, z0_ref):
    dinv = dinv_ref[...]
    axp = dinv * (S_ref[0] + S_ref[1]) + (dinv * dinv) * xp_ref[...]
    c1a_ref[...] = _mm(axp, wc_ref[...]) + bc_ref[...][None, :]
    xza_ref[...] = _mm(axp, wz_ref[...]) + bz_ref[...][None, :]
    xra_ref[...] = _mm(axp, wr_ref[...]) + br_ref[...][None, :]
    xha_ref[...] = _mm(axp, wh_ref[...]) + bh_ref[...][None, :]
    pr0 = jnp.maximum(bp_ref[...], 0.0)[None, :]
    z0row = _mm(pr0, wpm_ref[...]) + bpm_ref[...][None, :]
    z0_ref[...] = jnp.broadcast_to(z0row, (TB, ZD))


def _b0_body(c1a_ref, dinv_ref, wm_ref, y2_ref, y2s_ref):
    hc = jnp.maximum(c1a_ref[...], 0.0)
    y2 = _mm(hc, wm_ref[...])
    y2_ref[...] = y2
    y2s_ref[...] = y2 * dinv_ref[...]


def _b_body(S_ref, h_ref, dinv_ref, c1a_ref, wc_ref, wm_ref,
            whz_ref, bhz_ref, whr_ref, bhr_ref,
            y2_ref, y2s_ref, hz_ref, hr_ref):
    dinv = dinv_ref[...]
    h = h_ref[...]
    ah = dinv * (S_ref[0] + S_ref[1]) + (dinv * dinv) * h
    hc = jnp.maximum(c1a_ref[...] + _mm(ah, wc_ref[...]), 0.0)
    y2 = _mm(hc, wm_ref[...])
    y2_ref[...] = y2
    y2s_ref[...] = y2 * dinv
    hz_ref[...] = _mm(ah, whz_ref[...]) + bhz_ref[...][None, :]
    hr_ref[...] = _mm(ah, whr_ref[...]) + bhr_ref[...][None, :]


def _c_body(S_ref, y2_ref, dinv_ref, bm_ref, wpz_ref, bpz_ref,
            phi_ref, phis_ref):
    dinv = dinv_ref[...]
    z_enc = dinv * (S_ref[0] + S_ref[1]) \
        + (dinv * dinv) * y2_ref[...] + bm_ref[...][None, :]
    phi = jnp.maximum(_mm(z_enc, wpz_ref[...]) + bpz_ref[...][None, :], 0.0)
    phi_ref[...] = phi
    phis_ref[...] = phi * dinv


def _d0a_body(S_ref, phi_ref, dinv_ref, xza_ref, xha_ref,
              wzb_ref, whb_ref, bhz_ref, bhh_ref,
              wpr_ref, bpr_ref, wpm_ref, bpm_ref,
              h_ref, z_ref, hs_ref):
    dinv = dinv_ref[...]
    aphi = dinv * (S_ref[0] + S_ref[1]) + (dinv * dinv) * phi_ref[...]
    z_g = jax.nn.sigmoid(xza_ref[...] + _mm(aphi, wzb_ref[...])
                         + bhz_ref[...][None, :])
    xh = xha_ref[...] + _mm(aphi, whb_ref[...])
    h_hat = jnp.tanh(xh + bhh_ref[...][None, :])
    h = (1.0 - z_g) * h_hat
    h_ref[...] = h
    prior = jnp.maximum(_mm(h, wpr_ref[...]) + bpr_ref[...][None, :], 0.0)
    z_ref[...] = _mm(prior, wpm_ref[...]) + bpm_ref[...][None, :]
    hs_ref[...] = h * dinv


def _d_body(S_ref, phi_ref, dinv_ref, xza_ref, xra_ref, xha_ref,
            wzb_ref, wrb_ref, whb_ref, hz_ref, hr_ref, h_ref, whh_ref,
            zg_ref, xh_ref, y4_ref, y4s_ref):
    dinv = dinv_ref[...]
    aphi = dinv * (S_ref[0] + S_ref[1]) + (dinv * dinv) * phi_ref[...]
    z_g = jax.nn.sigmoid(xza_ref[...] + _mm(aphi, wzb_ref[...]) + hz_ref[...])
    r_g = jax.nn.sigmoid(xra_ref[...] + _mm(aphi, wrb_ref[...]) + hr_ref[...])
    zg_ref[...] = z_g
    xh_ref[...] = xha_ref[...] + _mm(aphi, whb_ref[...])
    y4 = _mm(r_g * h_ref[...], whh_ref[...])
    y4_ref[...] = y4
    y4s_ref[...] = y4 * dinv


def _ea_body(S_ref, y4_ref, dinv_ref, bhh_ref, xh_ref, zg_ref, hold_ref,
             wpr_ref, bpr_ref, wpm_ref, bpm_ref,
             h_ref, z_ref, hs_ref):
    dinv = dinv_ref[...]
    hh = dinv * (S_ref[0] + S_ref[1]) \
        + (dinv * dinv) * y4_ref[...] + bhh_ref[...][None, :]
    h_hat = jnp.tanh(xh_ref[...] + hh)
    z_g = zg_ref[...]
    h = z_g * hold_ref[0:1, :] + (1.0 - z_g) * h_hat
    h_ref[...] = h
    prior = jnp.maximum(_mm(h, wpr_ref[...]) + bpr_ref[...][None, :], 0.0)
    z_ref[...] = _mm(prior, wpm_ref[...]) + bpm_ref[...][None, :]
    hs_ref[...] = h * dinv


def _e_body(S_ref, y4_ref, dinv_ref, bhh_ref, xh_ref, zg_ref, hold_ref,
            h_ref):
    dinv = dinv_ref[...]
    hh = dinv * (S_ref[0] + S_ref[1]) \
        + (dinv * dinv) * y4_ref[...] + bhh_ref[...][None, :]
    h_hat = jnp.tanh(xh_ref[...] + hh)
    z_g = zg_ref[...]
    h_ref[...] = z_g * hold_ref[0:1, :] + (1.0 - z_g) * h_hat


_f32 = jnp.float32


def _sds(*shape):
    return jax.ShapeDtypeStruct(shape, _f32)


def kernel(xs, edge_index, W_phi_x, b_phi_x, W_prior, b_prior, W_pmean,
           b_pmean, W_pstd, b_pstd, W_c1, b_c1, W_mean, b_mean, W_phi_z,
           b_phi_z, W_xz, b_xz, W_hz, b_hz, W_xr, b_xr, W_hr, b_hr, W_xh,
           b_xh, W_hh, b_hh):
    s = edge_index[0]
    d = edge_index[1]
    pad = EPAD - E
    s3 = jnp.concatenate([s, jnp.zeros((pad,), jnp.int32)]) \
        .reshape(NTILES, CPT, CHUNK)
    d3 = jnp.concatenate([d, jnp.full((pad,), N, jnp.int32)]) \
        .reshape(NTILES, CPT, CHUNK)

    W_c1a, W_c1b = W_c1[:HD], W_c1[HD:]
    W_xza, W_xzb = W_xz[:HD], W_xz[HD:]
    W_xra, W_xrb = W_xr[:HD], W_xr[HD:]
    W_xha, W_xhb = W_xh[:HD], W_xh[HD:]

    w64 = _whole((HD, HD))
    w6432 = _whole((HD, ZD))
    w3264 = _whole((ZD, HD))
    b64 = _whole((HD,))
    b32 = _whole((ZD,))

    degp = _sc_deg(d3).reshape(NCORES, NP, 1)
    dinv, xp, xps = _tc(
        _prep_body, [_sds(N, 1), _sds(N, HD), _sds(N, HD)],
        [_Ssp(1), _row(XD), _whole((XD, HD)), b64],
        [_row(1), _row(HD), _row(HD)],
        degp, xs, W_phi_x, b_phi_x)

    Sxp = _sc_scatter64(xps, s3, d3)
    C1a, XZa, XRa, XHa, z0 = _tc(
        _axp_body,
        [_sds(N, HD), _sds(N, HD), _sds(N, HD), _sds(N, HD), _sds(N, ZD)],
        [_Ssp(HD), _row(HD), _row(1), w64, b64, w64, b64, w64, b64,
         w64, b64, b64, w6432, b32],
        [_row(HD), _row(HD), _row(HD), _row(HD), _row(ZD)],
        Sxp, xp, dinv, W_c1a, b_c1, W_xza, b_xz, W_xra, b_xr, W_xha, b_xh,
        b_prior, W_pmean, b_pmean)

    y2, y2s = _tc(
        _b0_body, [_sds(N, ZD), _sds(N, ZD)],
        [_row(HD), _row(1), w6432],
        [_row(ZD), _row(ZD)],
        C1a, dinv, W_mean)
    Sz = _sc_scatter32(y2s, s3, d3)
    phi, phis = _tc(
        _c_body, [_sds(N, HD), _sds(N, HD)],
        [_Ssp(ZD), _row(ZD), _row(1), b32, w3264, b64],
        [_row(HD), _row(HD)],
        Sz, y2, dinv, b_mean, W_phi_z, b_phi_z)
    Sphi = _sc_scatter64(phis, s3, d3)
    h, z1, hs = _tc(
        _d0a_body, [_sds(N, HD), _sds(N, ZD), _sds(N, HD)],
        [_Ssp(HD), _row(HD), _row(1), _row(HD), _row(HD),
         w64, w64, b64, b64, w64, b64, w6432, b32],
        [_row(HD), _row(ZD), _row(HD)],
        Sphi, phi, dinv, XZa, XHa, W_xzb, W_xhb, b_hz, b_hh,
        W_prior, b_prior, W_pmean, b_pmean)

    zs = [z0, z1]
    for t in (1, 2):
        Sh = _sc_scatter64(hs, s3, d3)
        y2, y2s, hz, hr = _tc(
            _b_body, [_sds(N, ZD), _sds(N, ZD), _sds(N, HD), _sds(N, HD)],
            [_Ssp(HD), _row(HD), _row(1), _row(HD), w64, w6432,
             w64, b64, w64, b64],
            [_row(ZD), _row(ZD), _row(HD), _row(HD)],
            Sh, h, dinv, C1a, W_c1b, W_mean, W_hz, b_hz, W_hr, b_hr)
        Sz = _sc_scatter32(y2s, s3, d3)
        phi, phis = _tc(
            _c_body, [_sds(N, HD), _sds(N, HD)],
            [_Ssp(ZD), _row(ZD), _row(1), b32, w3264, b64],
            [_row(HD), _row(HD)],
            Sz, y2, dinv, b_mean, W_phi_z, b_phi_z)
        Sphi = _sc_scatter64(phis, s3, d3)
        zg, xh, y4, y4s = _tc(
            _d_body, [_sds(N, HD), _sds(N, HD), _sds(N, HD), _sds(N, HD)],
            [_Ssp(HD), _row(HD), _row(1), _row(HD), _row(HD), _row(HD),
             w64, w64, w64, _row(HD), _row(HD), _row(HD), w64],
            [_row(HD), _row(HD), _row(HD), _row(HD)],
            Sphi, phi, dinv, XZa, XRa, XHa, W_xzb, W_xrb, W_xhb,
            hz, hr, h, W_hh)
        Su = _sc_scatter64(y4s, s3, d3)
        if t < 2:
            h, z_t, hs = _tc(
                _ea_body, [_sds(N, HD), _sds(N, ZD), _sds(N, HD)],
                [_Ssp(HD), _row(HD), _row(1), b64, _row(HD), _row(HD),
                 _row0(HD), w64, b64, w6432, b32],
                [_row(HD), _row(ZD), _row(HD)],
                Su, y4, dinv, b_hh, xh, zg, h,
                W_prior, b_prior, W_pmean, b_pmean)
            zs.append(z_t)
        else:
            h = _tc(
                _e_body, _sds(N, HD),
                [_Ssp(HD), _row(HD), _row(1), b64, _row(HD), _row(HD),
                 _row0(HD)],
                _row(HD),
                Su, y4, dinv, b_hh, xh, zg, h)

    return jnp.stack(zs), h
